# Initial kernel scaffold; baseline (speedup 1.0000x reference)
import jax
import jax.numpy as jnp
from jax.experimental import pallas as pl


def kernel(x, edge_index, edge_attr, W_l, b_l, W_r, b_r, W_e, att, bias):
    raise NotImplementedError("write your pallas kernel here")


# speedup vs baseline: 5.0206x; 5.0206x over previous
import functools

import jax
import jax.numpy as jnp
import numpy as np
from jax import lax
from jax.experimental import pallas as pl
from jax.experimental.pallas import tpu as pltpu
from jax.experimental.pallas import tpu_sc as plsc

NC = 2
NS = 16
L = 16
NW = NC * NS

C = 80
DSPAN = 2000
RSPAN = 624
RP = 104


def _lane_sum(v):
    fold = v + lax.rev(v, (0,))
    s = fold[0]
    for i in range(1, 8):
        s = s + fold[i]
    return s


def _lin_tc_kernel(x_ref, wl_ref, bl_ref, wr_ref, br_ref, xl_ref, xr_ref):
    xv = x_ref[...]
    xl_ref[...] = jnp.dot(xv, wl_ref[...],
                          preferred_element_type=jnp.float32) + bl_ref[...]
    xr_ref[...] = jnp.dot(xv, wr_ref[...],
                          preferred_element_type=jnp.float32) + br_ref[...]


def _combine_tc_kernel(p_ref, b_ref, o_ref):
    o_ref[...] = jnp.maximum(p_ref[...] + b_ref[...], 0.0)


def _densum_tc_kernel(p_ref, o_ref):
    o_ref[...] = p_ref[0:1, :] + p_ref[1:2, :]


def _edge_logits_body(n, d, xl_hbm, xr_hbm, src3, dst3, ea3, we_hbm, att_hbm,
                      ex3, den_hbm,
                      idx_s, idx_d, ea_v, ex_v, xl_rows, xr_rows, we_v, att_v,
                      zbuf, den_sh, sem1, sem2):
    dg = d // L
    nch = src3.shape[1]
    c_id = lax.axis_index("c")
    s_id = lax.axis_index("s")
    wid = s_id * NC + c_id

    def _z(t, _):
        zbuf[pl.ds(t * L, L)] = jnp.zeros((L,), jnp.float32)
        return 0
    lax.fori_loop(0, DSPAN // L, _z, 0)

    @pl.when(s_id < n // DSPAN)
    def _():
        pltpu.sync_copy(zbuf, den_sh.at[pl.ds(s_id * DSPAN, DSPAN)])
    plsc.subcore_barrier()

    pltpu.sync_copy(we_hbm, we_v)
    pltpu.sync_copy(att_hbm, att_v)
    pltpu.sync_copy(src3.at[wid], idx_s)
    pltpu.sync_copy(dst3.at[wid], idx_d)
    pltpu.sync_copy(ea3.at[wid], ea_v)

    def chunk(c, _):
        cp1 = pltpu.async_copy(xl_hbm.at[idx_s.at[c]], xl_rows, sem1)
        cp2 = pltpu.async_copy(xr_hbm.at[idx_d.at[c]], xr_rows, sem2)
        cp1.wait()
        cp2.wait()

        def group(g, _):
            joff = c * C + g * L
            ea0 = ea_v[0, pl.ds(joff, L)]
            ea1 = ea_v[1, pl.ds(joff, L)]
            ea2 = ea_v[2, pl.ds(joff, L)]
            ea3v = ea_v[3, pl.ds(joff, L)]
            lane = lax.broadcasted_iota(jnp.int32, (L,), 0)
            acc16 = jnp.zeros((L,), jnp.float32)
            for jj in range(L):
                a0, a1, a2, a3 = ea0[jj], ea1[jj], ea2[jj], ea3v[jj]
                jl = g * L + jj
                accv = jnp.zeros((L,), jnp.float32)
                for k in range(dg):
                    sl = pl.ds(k * L, L)
                    ef = (a0 * we_v[0, sl] + a1 * we_v[1, sl]
                          + a2 * we_v[2, sl] + a3 * we_v[3, sl])
                    h = xl_rows[jl, sl] + xr_rows[jl, sl] + ef
                    h = jnp.where(h > 0.0, h, 0.2 * h)
                    accv = accv + h * att_v[0, sl]
                acc16 = jnp.where(lane == jj, _lane_sum(accv), acc16)
            ex_v[c, pl.ds(g * L, L)] = jnp.exp(acc16)
            return 0
        lax.fori_loop(0, C // L, group, 0)
        pltpu.sync_copy(ex_v.at[c], den_sh.at[idx_d.at[c]], add=True)
        return 0
    lax.fori_loop(0, nch, chunk, 0)
    pltpu.sync_copy(ex_v, ex3.at[wid])

    plsc.subcore_barrier()

    @pl.when(s_id < n // DSPAN)
    def _():
        off = s_id * DSPAN
        pltpu.sync_copy(den_sh.at[pl.ds(off, DSPAN)], zbuf)
        pltpu.sync_copy(zbuf, den_hbm.at[pl.ds(c_id * n + off, DSPAN)])


def _edge_aggregate_body(n, d, xl_hbm, src2, dst2, ex2, den_hbm,
                         part_hbm,
                         idx_s, idx_d, ex_v, xl_rows, den_tot,
                         zrow, out_sh, sem1):
    dg = d // L
    nch = src2.shape[1]
    half = n // NC
    rspan = half // NS // 8 * 8
    rem = half - rspan * NS
    c_id = lax.axis_index("c")
    s_id = lax.axis_index("s")
    base_r = s_id * rspan
    lo = c_id * half

    pltpu.sync_copy(den_hbm, den_tot)

    rp = zrow.shape[0]

    def _zr(r, _):
        for k in range(dg):
            zrow[r, pl.ds(k * L, L)] = jnp.zeros((L,), jnp.float32)
        return 0
    lax.fori_loop(0, rp, _zr, 0)
    for q in range(rspan // rp):
        pltpu.sync_copy(zrow, out_sh.at[pl.ds(base_r + q * rp, rp)])

    @pl.when(s_id == NS - 1)
    def _():
        pltpu.sync_copy(zrow.at[pl.ds(0, rem + 8)],
                        out_sh.at[pl.ds(rspan * NS, rem + 8)])
    plsc.subcore_barrier()
    scs = idx_s.shape[0]

    def superchunk(ss, _):
        pltpu.sync_copy(src2.at[s_id, pl.ds(ss * scs, scs)], idx_s)
        pltpu.sync_copy(dst2.at[s_id, pl.ds(ss * scs, scs)], idx_d)
        pltpu.sync_copy(ex2.at[s_id, pl.ds(ss * scs, scs)], ex_v)

        def chunk(c, _):
            pltpu.async_copy(xl_hbm.at[idx_s.at[c]], xl_rows, sem1).wait()

            def group(g, _):
                sl16 = pl.ds(g * L, L)
                idx16 = idx_d[c, sl16]
                ex16 = ex_v[c, sl16]
                den16 = plsc.load_gather(den_tot, [idx16])
                al16 = ex16 / (den16 + 1e-16)
                loc16 = idx16 - lo
                inr = (loc16 >= 0) & (loc16 < half)
                idx_d[c, sl16] = jnp.where(inr, loc16, half)
                for jj in range(L):
                    a = al16[jj]
                    jl = g * L + jj
                    for k in range(dg):
                        sl = pl.ds(k * L, L)
                        xl_rows[jl, sl] = xl_rows[jl, sl] * a
                return 0
            lax.fori_loop(0, C // L, group, 0)
            pltpu.sync_copy(xl_rows, out_sh.at[idx_d.at[c]], add=True)
            return 0
        lax.fori_loop(0, scs, chunk, 0)
        return 0
    lax.fori_loop(0, nch // scs, superchunk, 0)

    plsc.subcore_barrier()
    for q in range(rspan // rp):
        r0 = base_r + q * rp
        pltpu.sync_copy(out_sh.at[pl.ds(r0, rp)], zrow)
        pltpu.sync_copy(zrow, part_hbm.at[pl.ds(lo + r0, rp)])

    @pl.when(s_id == NS - 1)
    def _():
        r0 = rspan * NS
        pltpu.sync_copy(out_sh.at[pl.ds(r0, rem)], zrow.at[pl.ds(0, rem)])
        pltpu.sync_copy(zrow.at[pl.ds(0, rem)],
                        part_hbm.at[pl.ds(lo + r0, rem)])


def kernel(x, edge_index, edge_attr, W_l, b_l, W_r, b_r, W_e, att, bias):
    n, d = x.shape
    e = edge_index.shape[1]
    de = edge_attr.shape[1]
    assert e % (NW * C) == 0
    assert n % DSPAN == 0 and n % L == 0 and d % L == 0
    assert RSPAN % RP == 0 and 0 < n - RSPAN * NS <= RP
    epw = e // NW
    nch = epw // C

    src3 = edge_index[0].astype(jnp.int32).reshape(NW, nch, C)
    dst3 = edge_index[1].astype(jnp.int32).reshape(NW, nch, C)
    ea3 = edge_attr.reshape(NW, epw, de).transpose(0, 2, 1)
    b_l2 = b_l.reshape(1, d)
    b_r2 = b_r.reshape(1, d)
    bias2 = bias.reshape(1, d)
    att2 = att.reshape(1, d)

    rb = 400
    grid = n // rb
    xl, xr = pl.pallas_call(
        _lin_tc_kernel,
        grid=(grid,),
        in_specs=[
            pl.BlockSpec((rb, d), lambda i: (i, 0)),
            pl.BlockSpec((d, d), lambda i: (0, 0)),
            pl.BlockSpec((1, d), lambda i: (0, 0)),
            pl.BlockSpec((d, d), lambda i: (0, 0)),
            pl.BlockSpec((1, d), lambda i: (0, 0)),
        ],
        out_specs=[
            pl.BlockSpec((rb, d), lambda i: (i, 0)),
            pl.BlockSpec((rb, d), lambda i: (i, 0)),
        ],
        out_shape=[
            jax.ShapeDtypeStruct((n, d), jnp.float32),
            jax.ShapeDtypeStruct((n, d), jnp.float32),
        ],
    )(x, W_l, b_l2, W_r, b_r2)

    mesh = plsc.VectorSubcoreMesh(core_axis_name="c", subcore_axis_name="s",
                                  num_cores=NC, num_subcores=NS)
    sc_params = pltpu.CompilerParams(needs_layout_passes=False)

    sc1 = pl.kernel(
        functools.partial(_edge_logits_body, n, d),
        out_type=[
            jax.ShapeDtypeStruct((NW, nch, C), jnp.float32),
            jax.ShapeDtypeStruct((NC * n,), jnp.float32),
        ],
        mesh=mesh,
        scratch_types=[
            pltpu.VMEM((nch, C), jnp.int32),
            pltpu.VMEM((nch, C), jnp.int32),
            pltpu.VMEM((de, epw), jnp.float32),
            pltpu.VMEM((nch, C), jnp.float32),
            pltpu.VMEM((C, d), jnp.float32),
            pltpu.VMEM((C, d), jnp.float32),
            pltpu.VMEM((de, d), jnp.float32),
            pltpu.VMEM((1, d), jnp.float32),
            pltpu.VMEM((DSPAN,), jnp.float32),
            pltpu.VMEM_SHARED((n,), jnp.float32),
            pltpu.SemaphoreType.DMA,
            pltpu.SemaphoreType.DMA,
        ],
        compiler_params=sc_params,
    )
    ex3, denp = sc1(xl, xr, src3, dst3, ea3, W_e, att2)

    dentot = pl.pallas_call(
        _densum_tc_kernel,
        in_specs=[pl.BlockSpec((NC, n), lambda: (0, 0))],
        out_specs=pl.BlockSpec((1, n), lambda: (0, 0)),
        out_shape=jax.ShapeDtypeStruct((1, n), jnp.float32),
    )(denp.reshape(NC, n)).reshape(n)

    nch2 = -(-e // (NS * C * 8)) * 8
    e2 = NS * nch2 * C
    src2 = jnp.pad(edge_index[0].astype(jnp.int32),
                   (0, e2 - e)).reshape(NS, nch2, C)
    dst2 = jnp.pad(edge_index[1].astype(jnp.int32),
                   (0, e2 - e)).reshape(NS, nch2, C)
    ex2 = jnp.pad(ex3.reshape(-1), (0, e2 - e)).reshape(NS, nch2, C)
    half = n // NC

    sc2 = pl.kernel(
        functools.partial(_edge_aggregate_body, n, d),
        out_type=[
            jax.ShapeDtypeStruct((n, d), jnp.float32),
        ],
        mesh=mesh,
        scratch_types=[
            pltpu.VMEM((8, C), jnp.int32),
            pltpu.VMEM((8, C), jnp.int32),
            pltpu.VMEM((8, C), jnp.float32),
            pltpu.VMEM((C, d), jnp.float32),
            pltpu.VMEM((n,), jnp.float32),
            pltpu.VMEM((24, d), jnp.float32),
            pltpu.VMEM_SHARED((half + 8, d), jnp.float32),
            pltpu.SemaphoreType.DMA,
        ],
        compiler_params=sc_params,
    )
    (part,) = sc2(xl, src2, dst2, ex2, dentot)

    out = pl.pallas_call(
        _combine_tc_kernel,
        grid=(grid,),
        in_specs=[
            pl.BlockSpec((rb, d), lambda i: (i, 0)),
            pl.BlockSpec((1, d), lambda i: (0, 0)),
        ],
        out_specs=pl.BlockSpec((rb, d), lambda i: (i, 0)),
        out_shape=jax.ShapeDtypeStruct((n, d), jnp.float32),
    )(part, bias2)
    return out
